# Initial kernel scaffold; baseline (speedup 1.0000x reference)
import jax
import jax.numpy as jnp
from jax.experimental import pallas as pl


def kernel(bbox, x_table, y_table, h_table, w_table):
    raise NotImplementedError("write your pallas kernel here")


# speedup vs baseline: 3.6237x; 3.6237x over previous
import functools

import jax
import jax.numpy as jnp
from jax import lax
from jax.experimental import pallas as pl
from jax.experimental.pallas import tpu as pltpu
from jax.experimental.pallas import tpu_sc as plsc

HIDDEN = 128
MAX_POS = 1000
NUM_K = 4

_info = plsc.get_sparse_core_info()
_NC, _NS, _L = _info.num_cores, _info.num_subcores, _info.num_lanes
_NW = _NC * _NS

N_ROWS = 1024 * 200
ROWS_PER_W = N_ROWS // _NW
CHUNK = 128
N_CHUNKS = ROWS_PER_W // CHUNK
IDX_PER_CHUNK = CHUNK * NUM_K


def _sc_body(table_hbm, bbox_hbm, out_hbm, idx_v, buf_v, out_v, sem):
    wid = lax.axis_index("s") * _NC + lax.axis_index("c")
    pat = lax.rem(lax.iota(jnp.int32, _L), NUM_K) * MAX_POS

    def chunk_body(c, _):
        row_base = wid * ROWS_PER_W + c * CHUNK
        flat_base = row_base * NUM_K

        pltpu.sync_copy(bbox_hbm.at[pl.ds(flat_base, IDX_PER_CHUNK)], idx_v)

        for g in range(IDX_PER_CHUNK // _L):
            sl = pl.ds(g * _L, _L)
            idx_v[sl] = idx_v[sl] + pat

        copies = [
            pltpu.async_copy(
                table_hbm.at[idx_v.at[pl.ds(j * CHUNK, CHUNK)]],
                buf_v.at[pl.ds(j * CHUNK, CHUNK)],
                sem,
            )
            for j in range(NUM_K)
        ]
        for cp in copies:
            cp.wait()

        def row_body(r, _):
            g = r * NUM_K
            for u in range(HIDDEN // _L):
                sl = pl.ds(u * _L, _L)
                out_v[r, sl] = (
                    (buf_v[g, sl] + buf_v[g + 1, sl])
                    + (buf_v[g + 2, sl] + buf_v[g + 3, sl])
                )
            return 0

        lax.fori_loop(0, CHUNK, row_body, 0, unroll=False)

        pltpu.sync_copy(out_v, out_hbm.at[pl.ds(row_base, CHUNK)])
        return 0

    lax.fori_loop(0, N_CHUNKS, chunk_body, 0, unroll=False)


@jax.jit
def kernel(bbox, x_table, y_table, h_table, w_table):
    table = jnp.concatenate([x_table, y_table, h_table, w_table], axis=0)
    bbox_flat = bbox.astype(jnp.int32).reshape(N_ROWS * NUM_K)

    mesh = plsc.VectorSubcoreMesh(core_axis_name="c", subcore_axis_name="s")
    run = pl.kernel(
        _sc_body,
        out_type=jax.ShapeDtypeStruct((N_ROWS, HIDDEN), jnp.float32),
        mesh=mesh,
        scratch_types=[
            pltpu.VMEM((IDX_PER_CHUNK,), jnp.int32),
            pltpu.VMEM((IDX_PER_CHUNK, HIDDEN), jnp.float32),
            pltpu.VMEM((CHUNK, HIDDEN), jnp.float32),
            pltpu.SemaphoreType.DMA,
        ],
    )
    out = run(table, bbox_flat)
    return out.reshape(1024, 200, HIDDEN)


# speedup vs baseline: 3.6321x; 1.0023x over previous
import functools

import jax
import jax.numpy as jnp
from jax import lax
from jax.experimental import pallas as pl
from jax.experimental.pallas import tpu as pltpu
from jax.experimental.pallas import tpu_sc as plsc

HIDDEN = 128
MAX_POS = 1000
NUM_K = 4

_info = plsc.get_sparse_core_info()
_NC, _NS, _L = _info.num_cores, _info.num_subcores, _info.num_lanes
_NW = _NC * _NS

N_ROWS = 1024 * 200
ROWS_PER_W = N_ROWS // _NW
CHUNK = 128
N_CHUNKS = ROWS_PER_W // CHUNK
IDX_PER_CHUNK = CHUNK * NUM_K


def _sc_body(table_hbm, bbox_hbm, out_hbm, idx_v, buf_v, out_v, sem):
    wid = lax.axis_index("s") * _NC + lax.axis_index("c")
    pat = lax.rem(lax.iota(jnp.int32, _L), NUM_K) * MAX_POS

    def chunk_body(c, _):
        row_base = wid * ROWS_PER_W + c * CHUNK
        flat_base = row_base * NUM_K

        pltpu.sync_copy(bbox_hbm.at[pl.ds(flat_base, IDX_PER_CHUNK)], idx_v)

        for g in range(IDX_PER_CHUNK // _L):
            sl = pl.ds(g * _L, _L)
            idx_v[sl] = idx_v[sl] + pat

        pltpu.async_copy(table_hbm.at[idx_v], buf_v, sem).wait()

        def row_body(r, _):
            g = r * NUM_K
            for u in range(HIDDEN // _L):
                sl = pl.ds(u * _L, _L)
                out_v[r, sl] = (
                    (buf_v[g, sl] + buf_v[g + 1, sl])
                    + (buf_v[g + 2, sl] + buf_v[g + 3, sl])
                )
            return 0

        lax.fori_loop(0, CHUNK, row_body, 0, unroll=False)

        pltpu.sync_copy(out_v, out_hbm.at[pl.ds(row_base, CHUNK)])
        return 0

    lax.fori_loop(0, N_CHUNKS, chunk_body, 0, unroll=False)


@jax.jit
def kernel(bbox, x_table, y_table, h_table, w_table):
    table = jnp.concatenate([x_table, y_table, h_table, w_table], axis=0)
    bbox_flat = bbox.astype(jnp.int32).reshape(N_ROWS * NUM_K)

    mesh = plsc.VectorSubcoreMesh(core_axis_name="c", subcore_axis_name="s")
    run = pl.kernel(
        _sc_body,
        out_type=jax.ShapeDtypeStruct((N_ROWS, HIDDEN), jnp.float32),
        mesh=mesh,
        scratch_types=[
            pltpu.VMEM((IDX_PER_CHUNK,), jnp.int32),
            pltpu.VMEM((IDX_PER_CHUNK, HIDDEN), jnp.float32),
            pltpu.VMEM((CHUNK, HIDDEN), jnp.float32),
            pltpu.SemaphoreType.DMA,
        ],
    )
    out = run(table, bbox_flat)
    return out.reshape(1024, 200, HIDDEN)


# speedup vs baseline: 4.7639x; 1.3116x over previous
import jax
import jax.numpy as jnp
from jax import lax
from jax.experimental import pallas as pl
from jax.experimental.pallas import tpu as pltpu
from jax.experimental.pallas import tpu_sc as plsc

HIDDEN = 128
MAX_POS = 1000
NUM_K = 4

_info = plsc.get_sparse_core_info()
_NC, _NS, _L = _info.num_cores, _info.num_subcores, _info.num_lanes
_NW = _NC * _NS

N_ROWS = 1024 * 200
ROWS_PER_W = N_ROWS // _NW
CHUNK = 64
N_CHUNKS = ROWS_PER_W // CHUNK
IDX_PER_CHUNK = CHUNK * NUM_K


def _sc_body(table_hbm, bbox_hbm, out_hbm,
             idx0, idx1, buf0, buf1, outv0, outv1,
             gsem0, gsem1, osem0, osem1):
    wid = lax.axis_index("s") * _NC + lax.axis_index("c")
    base_row_w = wid * ROWS_PER_W
    pat = lax.rem(lax.iota(jnp.int32, _L), NUM_K) * MAX_POS
    idxs = (idx0, idx1)
    bufs = (buf0, buf1)
    outs = (outv0, outv1)
    gsems = (gsem0, gsem1)
    osems = (osem0, osem1)

    def stage(c, b):
        flat_base = (base_row_w + c * CHUNK) * NUM_K
        pltpu.sync_copy(bbox_hbm.at[pl.ds(flat_base, IDX_PER_CHUNK)], idxs[b])
        for g in range(IDX_PER_CHUNK // _L):
            sl = pl.ds(g * _L, _L)
            idxs[b][sl] = idxs[b][sl] + pat
        pltpu.async_copy(table_hbm.at[idxs[b]], bufs[b], gsems[b])

    def gather_wait(b):
        pltpu.make_async_copy(table_hbm.at[idxs[b]], bufs[b], gsems[b]).wait()

    def store(c, b):
        row_base = base_row_w + c * CHUNK
        pltpu.async_copy(outs[b], out_hbm.at[pl.ds(row_base, CHUNK)], osems[b])

    def store_wait(b):
        pltpu.make_async_copy(outs[b], out_hbm.at[pl.ds(0, CHUNK)],
                              osems[b]).wait()

    def compute(b):
        buf, outv = bufs[b], outs[b]

        def row_body(r, _):
            g = r * NUM_K
            for u in range(HIDDEN // _L):
                sl = pl.ds(u * _L, _L)
                outv[r, sl] = (
                    (buf[g, sl] + buf[g + 1, sl])
                    + (buf[g + 2, sl] + buf[g + 3, sl])
                )
            return 0

        lax.fori_loop(0, CHUNK, row_body, 0, unroll=False)

    stage(0, 0)
    stage(1, 1)
    for b in range(2):
        gather_wait(b)
        compute(b)
        store(b, b)
        stage(b + 2, b)

    def pair_body(p, _):
        for b in range(2):
            c = p * 2 + b
            gather_wait(b)
            store_wait(b)
            compute(b)
            store(c, b)
            stage(c + 2, b)
        return 0

    lax.fori_loop(1, N_CHUNKS // 2 - 1, pair_body, 0, unroll=False)

    for b in range(2):
        c = N_CHUNKS - 2 + b
        gather_wait(b)
        store_wait(b)
        compute(b)
        store(c, b)
    for b in range(2):
        store_wait(b)


@jax.jit
def kernel(bbox, x_table, y_table, h_table, w_table):
    table = jnp.concatenate([x_table, y_table, h_table, w_table], axis=0)
    bbox_flat = bbox.astype(jnp.int32).reshape(N_ROWS * NUM_K)

    mesh = plsc.VectorSubcoreMesh(core_axis_name="c", subcore_axis_name="s")
    run = pl.kernel(
        _sc_body,
        out_type=jax.ShapeDtypeStruct((N_ROWS, HIDDEN), jnp.float32),
        mesh=mesh,
        scratch_types=[
            pltpu.VMEM((IDX_PER_CHUNK,), jnp.int32),
            pltpu.VMEM((IDX_PER_CHUNK,), jnp.int32),
            pltpu.VMEM((IDX_PER_CHUNK, HIDDEN), jnp.float32),
            pltpu.VMEM((IDX_PER_CHUNK, HIDDEN), jnp.float32),
            pltpu.VMEM((CHUNK, HIDDEN), jnp.float32),
            pltpu.VMEM((CHUNK, HIDDEN), jnp.float32),
            pltpu.SemaphoreType.DMA,
            pltpu.SemaphoreType.DMA,
            pltpu.SemaphoreType.DMA,
            pltpu.SemaphoreType.DMA,
        ],
    )
    out = run(table, bbox_flat)
    return out.reshape(1024, 200, HIDDEN)


# speedup vs baseline: 10.3308x; 2.1685x over previous
import jax
import jax.numpy as jnp
from jax import lax
from jax.experimental import pallas as pl
from jax.experimental.pallas import tpu as pltpu
from jax.experimental.pallas import tpu_sc as plsc

HIDDEN = 128
MAX_POS = 1000
NUM_K = 4

_info = plsc.get_sparse_core_info()
_NC, _NS, _L = _info.num_cores, _info.num_subcores, _info.num_lanes
_NW = _NC * _NS

N_ROWS = 1024 * 200
ROWS_PER_W = N_ROWS // _NW
CHUNK = 128
N_CHUNKS = ROWS_PER_W // CHUNK
N_PAIRS = N_CHUNKS // 2


def _sc_body(xt_hbm, yt_hbm, ht_hbm, wt_hbm, bboxw_hbm, out_hbm,
             idx_all, outv0, outv1, gsem0, gsem1, osem0, osem1):
    wid = lax.axis_index("s") * _NC + lax.axis_index("c")
    base_row_w = wid * ROWS_PER_W
    tabs = (xt_hbm, yt_hbm, ht_hbm, wt_hbm)
    outs = (outv0, outv1)
    gsems = (gsem0, gsem1)
    osems = (osem0, osem1)

    pltpu.sync_copy(bboxw_hbm.at[wid], idx_all)

    def fire(c, b, k):
        pltpu.async_copy(tabs[k].at[idx_all.at[k * N_CHUNKS + c]],
                         outs[b], gsems[b], add=(k > 0))

    def gather_wait(b):
        pltpu.make_async_copy(tabs[0].at[idx_all.at[0]], outs[b],
                              gsems[b]).wait()

    def store(c, b):
        row_base = base_row_w + c * CHUNK
        pltpu.async_copy(outs[b], out_hbm.at[pl.ds(row_base, CHUNK)],
                         osems[b])

    def store_wait(b):
        pltpu.make_async_copy(outs[b], out_hbm.at[pl.ds(0, CHUNK)],
                              osems[b]).wait()

    def pair(p, with_store_wait):
        c0 = p * 2
        c1 = c0 + 1
        if with_store_wait:
            store_wait(0)
            store_wait(1)
        fire(c0, 0, 0)
        fire(c1, 1, 0)
        for k in range(1, NUM_K):
            gather_wait(0)
            fire(c0, 0, k)
            gather_wait(1)
            fire(c1, 1, k)
        gather_wait(0)
        store(c0, 0)
        gather_wait(1)
        store(c1, 1)

    pair(0, False)

    def pair_body(p, _):
        pair(p, True)
        return 0

    lax.fori_loop(1, N_PAIRS, pair_body, 0, unroll=False)

    store_wait(0)
    store_wait(1)


@jax.jit
def kernel(bbox, x_table, y_table, h_table, w_table):
    bboxw = (bbox.astype(jnp.int32)
             .transpose(2, 0, 1)
             .reshape(NUM_K, _NW, N_CHUNKS, CHUNK)
             .transpose(1, 0, 2, 3)
             .reshape(_NW, NUM_K * N_CHUNKS, CHUNK))

    mesh = plsc.VectorSubcoreMesh(core_axis_name="c", subcore_axis_name="s")
    run = pl.kernel(
        _sc_body,
        out_type=jax.ShapeDtypeStruct((N_ROWS, HIDDEN), jnp.float32),
        mesh=mesh,
        scratch_types=[
            pltpu.VMEM((NUM_K * N_CHUNKS, CHUNK), jnp.int32),
            pltpu.VMEM((CHUNK, HIDDEN), jnp.float32),
            pltpu.VMEM((CHUNK, HIDDEN), jnp.float32),
            pltpu.SemaphoreType.DMA,
            pltpu.SemaphoreType.DMA,
            pltpu.SemaphoreType.DMA,
            pltpu.SemaphoreType.DMA,
        ],
    )
    out = run(x_table, y_table, h_table, w_table, bboxw)
    return out.reshape(1024, 200, HIDDEN)


# speedup vs baseline: 10.3742x; 1.0042x over previous
import jax
import jax.numpy as jnp
from jax import lax
from jax.experimental import pallas as pl
from jax.experimental.pallas import tpu as pltpu
from jax.experimental.pallas import tpu_sc as plsc

HIDDEN = 128
MAX_POS = 1000
NUM_K = 4

_info = plsc.get_sparse_core_info()
_NC, _NS, _L = _info.num_cores, _info.num_subcores, _info.num_lanes
_NW = _NC * _NS

N_ROWS = 1024 * 200
ROWS_PER_W = N_ROWS // _NW
CHUNK = 128
N_CHUNKS = ROWS_PER_W // CHUNK
N_PAIRS = N_CHUNKS // 2


SLOTS = 4
N_FULL_ROUNDS = N_CHUNKS // SLOTS
N_LEFTOVER = N_CHUNKS - N_FULL_ROUNDS * SLOTS


def _sc_body(xt_hbm, yt_hbm, ht_hbm, wt_hbm, bboxw_hbm, out_hbm,
             idx_all, outv0, outv1, outv2, outv3,
             gsem0, gsem1, gsem2, gsem3,
             osem0, osem1, osem2, osem3):
    wid = lax.axis_index("s") * _NC + lax.axis_index("c")
    base_row_w = wid * ROWS_PER_W
    tabs = (xt_hbm, yt_hbm, ht_hbm, wt_hbm)
    outs = (outv0, outv1, outv2, outv3)
    gsems = (gsem0, gsem1, gsem2, gsem3)
    osems = (osem0, osem1, osem2, osem3)

    pltpu.sync_copy(bboxw_hbm.at[wid], idx_all)

    def fire(c, b, k):
        pltpu.async_copy(tabs[k].at[idx_all.at[k * N_CHUNKS + c]],
                         outs[b], gsems[b], add=(k > 0))

    def gather_wait(b):
        pltpu.make_async_copy(tabs[0].at[idx_all.at[0]], outs[b],
                              gsems[b]).wait()

    def store(c, b):
        row_base = base_row_w + c * CHUNK
        pltpu.async_copy(outs[b], out_hbm.at[pl.ds(row_base, CHUNK)],
                         osems[b])

    def store_wait(b):
        pltpu.make_async_copy(outs[b], out_hbm.at[pl.ds(0, CHUNK)],
                              osems[b]).wait()

    def round_(p, with_store_wait, nslots=SLOTS):
        cs = [p * SLOTS + b for b in range(nslots)]
        if with_store_wait:
            for b in range(nslots):
                store_wait(b)
        for b in range(nslots):
            fire(cs[b], b, 0)
        for k in range(1, NUM_K):
            for b in range(nslots):
                gather_wait(b)
                fire(cs[b], b, k)
        for b in range(nslots):
            gather_wait(b)
            store(cs[b], b)

    round_(0, False)

    def round_body(p, _):
        round_(p, True)
        return 0

    lax.fori_loop(1, N_FULL_ROUNDS, round_body, 0, unroll=False)

    if N_LEFTOVER:
        round_(N_FULL_ROUNDS, True, nslots=N_LEFTOVER)

    for b in range(SLOTS):
        store_wait(b)


@jax.jit
def kernel(bbox, x_table, y_table, h_table, w_table):
    bboxw = (bbox.astype(jnp.int32)
             .transpose(2, 0, 1)
             .reshape(NUM_K, _NW, N_CHUNKS, CHUNK)
             .transpose(1, 0, 2, 3)
             .reshape(_NW, NUM_K * N_CHUNKS, CHUNK))

    mesh = plsc.VectorSubcoreMesh(core_axis_name="c", subcore_axis_name="s")
    run = pl.kernel(
        _sc_body,
        out_type=jax.ShapeDtypeStruct((N_ROWS, HIDDEN), jnp.float32),
        mesh=mesh,
        scratch_types=[
            pltpu.VMEM((NUM_K * N_CHUNKS, CHUNK), jnp.int32),
        ] + [pltpu.VMEM((CHUNK, HIDDEN), jnp.float32)] * SLOTS
          + [pltpu.SemaphoreType.DMA] * (2 * SLOTS),
    )
    out = run(x_table, y_table, h_table, w_table, bboxw)
    return out.reshape(1024, 200, HIDDEN)


# speedup vs baseline: 14.6901x; 1.4160x over previous
import jax
import jax.numpy as jnp
from jax import lax
from jax.experimental import pallas as pl
from jax.experimental.pallas import tpu as pltpu
from jax.experimental.pallas import tpu_sc as plsc

HIDDEN = 128
MAX_POS = 1000
NUM_K = 4

_info = plsc.get_sparse_core_info()
_NC, _NS, _L = _info.num_cores, _info.num_subcores, _info.num_lanes
_NW = _NC * _NS

N_ROWS = 1024 * 200
ROWS_PER_W = N_ROWS // _NW
CHUNK = 128
N_CHUNKS = ROWS_PER_W // CHUNK
N_PAIRS = N_CHUNKS // 2


SLOTS = 4
N_FULL_ROUNDS = N_CHUNKS // SLOTS
N_LEFTOVER = N_CHUNKS - N_FULL_ROUNDS * SLOTS


def _sc_body(xt_hbm, yt_hbm, ht_hbm, wt_hbm, bboxw_hbm, out_hbm,
             idx_all, sh_tab, outv0, outv1, outv2, outv3,
             gsem0, gsem1, gsem2, gsem3,
             osem0, osem1, osem2, osem3):
    sid = lax.axis_index("s")
    wid = sid * _NC + lax.axis_index("c")
    base_row_w = wid * ROWS_PER_W
    hbm_tabs = (xt_hbm, yt_hbm, ht_hbm, wt_hbm)
    outs = (outv0, outv1, outv2, outv3)
    gsems = (gsem0, gsem1, gsem2, gsem3)
    osems = (osem0, osem1, osem2, osem3)

    @pl.when(sid == 0)
    def _():
        for k in range(NUM_K):
            pltpu.sync_copy(hbm_tabs[k],
                            sh_tab.at[pl.ds(k * MAX_POS, MAX_POS)])

    pltpu.sync_copy(bboxw_hbm.at[wid], idx_all)
    plsc.subcore_barrier()
    tabs = tuple(sh_tab.at[pl.ds(k * MAX_POS, MAX_POS)]
                 for k in range(NUM_K))

    def fire(c, b, k):
        pltpu.async_copy(tabs[k].at[idx_all.at[k * N_CHUNKS + c]],
                         outs[b], gsems[b], add=(k > 0))

    def gather_wait(b):
        pltpu.make_async_copy(tabs[0].at[idx_all.at[0]], outs[b],
                              gsems[b]).wait()

    def store(c, b):
        row_base = base_row_w + c * CHUNK
        pltpu.async_copy(outs[b], out_hbm.at[pl.ds(row_base, CHUNK)],
                         osems[b])

    def store_wait(b):
        pltpu.make_async_copy(outs[b], out_hbm.at[pl.ds(0, CHUNK)],
                              osems[b]).wait()

    def round_(p, with_store_wait, nslots=SLOTS):
        cs = [p * SLOTS + b for b in range(nslots)]
        if with_store_wait:
            for b in range(nslots):
                store_wait(b)
        for b in range(nslots):
            fire(cs[b], b, 0)
        for k in range(1, NUM_K):
            for b in range(nslots):
                gather_wait(b)
                fire(cs[b], b, k)
        for b in range(nslots):
            gather_wait(b)
            store(cs[b], b)

    round_(0, False)

    def round_body(p, _):
        round_(p, True)
        return 0

    lax.fori_loop(1, N_FULL_ROUNDS, round_body, 0, unroll=False)

    if N_LEFTOVER:
        round_(N_FULL_ROUNDS, True, nslots=N_LEFTOVER)

    for b in range(SLOTS):
        store_wait(b)


@jax.jit
def kernel(bbox, x_table, y_table, h_table, w_table):
    bboxw = (bbox.astype(jnp.int32)
             .transpose(2, 0, 1)
             .reshape(NUM_K, _NW, N_CHUNKS, CHUNK)
             .transpose(1, 0, 2, 3)
             .reshape(_NW, NUM_K * N_CHUNKS, CHUNK))

    mesh = plsc.VectorSubcoreMesh(core_axis_name="c", subcore_axis_name="s")
    run = pl.kernel(
        _sc_body,
        out_type=jax.ShapeDtypeStruct((N_ROWS, HIDDEN), jnp.float32),
        mesh=mesh,
        scratch_types=[
            pltpu.VMEM((NUM_K * N_CHUNKS, CHUNK), jnp.int32),
            pltpu.VMEM_SHARED((NUM_K * MAX_POS, HIDDEN), jnp.float32),
        ] + [pltpu.VMEM((CHUNK, HIDDEN), jnp.float32)] * SLOTS
          + [pltpu.SemaphoreType.DMA] * (2 * SLOTS),
    )
    out = run(x_table, y_table, h_table, w_table, bboxw)
    return out.reshape(1024, 200, HIDDEN)


# speedup vs baseline: 15.4209x; 1.0497x over previous
import jax
import jax.numpy as jnp
from jax import lax
from jax.experimental import pallas as pl
from jax.experimental.pallas import tpu as pltpu
from jax.experimental.pallas import tpu_sc as plsc

HIDDEN = 128
MAX_POS = 1000
NUM_K = 4

_info = plsc.get_sparse_core_info()
_NC, _NS, _L = _info.num_cores, _info.num_subcores, _info.num_lanes
_NW = _NC * _NS

N_ROWS = 1024 * 200
ROWS_PER_W = N_ROWS // _NW
CHUNK = 128
N_CHUNKS = ROWS_PER_W // CHUNK
N_PAIRS = N_CHUNKS // 2


SLOTS = 4
N_FULL_ROUNDS = N_CHUNKS // SLOTS
N_LEFTOVER = N_CHUNKS - N_FULL_ROUNDS * SLOTS


def _sc_body(xt_hbm, yt_hbm, ht_hbm, wt_hbm, bboxw_hbm, out_hbm,
             idx_all, sh_tab, outv0, outv1, outv2, outv3,
             gsem0, gsem1, gsem2, gsem3,
             osem0, osem1, osem2, osem3):
    sid = lax.axis_index("s")
    wid = sid * _NC + lax.axis_index("c")
    base_row_w = wid * ROWS_PER_W
    hbm_tabs = (xt_hbm, yt_hbm, ht_hbm, wt_hbm)
    outs = (outv0, outv1, outv2, outv3)
    gsems = (gsem0, gsem1, gsem2, gsem3)
    osems = (osem0, osem1, osem2, osem3)

    @pl.when(sid == 0)
    def _():
        for k in range(NUM_K):
            pltpu.sync_copy(hbm_tabs[k],
                            sh_tab.at[pl.ds(k * MAX_POS, MAX_POS)])

    pltpu.sync_copy(bboxw_hbm.at[wid], idx_all)
    plsc.subcore_barrier()
    sh_tabs = tuple(sh_tab.at[pl.ds(k * MAX_POS, MAX_POS)]
                    for k in range(NUM_K))
    tabs_by_slot = (sh_tabs, sh_tabs, hbm_tabs, hbm_tabs)

    def fire(c, b, k):
        pltpu.async_copy(tabs_by_slot[b][k].at[idx_all.at[k * N_CHUNKS + c]],
                         outs[b], gsems[b], add=(k > 0))

    def gather_wait(b):
        pltpu.make_async_copy(tabs_by_slot[b][0].at[idx_all.at[0]], outs[b],
                              gsems[b]).wait()

    def store(c, b):
        row_base = base_row_w + c * CHUNK
        pltpu.async_copy(outs[b], out_hbm.at[pl.ds(row_base, CHUNK)],
                         osems[b])

    def store_wait(b):
        pltpu.make_async_copy(outs[b], out_hbm.at[pl.ds(0, CHUNK)],
                              osems[b]).wait()

    def round_(p, with_store_wait, nslots=SLOTS):
        cs = [p * SLOTS + b for b in range(nslots)]
        if with_store_wait:
            for b in range(nslots):
                store_wait(b)
        for b in range(nslots):
            fire(cs[b], b, 0)
        for k in range(1, NUM_K):
            for b in range(nslots):
                gather_wait(b)
                fire(cs[b], b, k)
        for b in range(nslots):
            gather_wait(b)
            store(cs[b], b)

    round_(0, False)

    def round_body(p, _):
        round_(p, True)
        return 0

    lax.fori_loop(1, N_FULL_ROUNDS, round_body, 0, unroll=False)

    if N_LEFTOVER:
        round_(N_FULL_ROUNDS, True, nslots=N_LEFTOVER)

    for b in range(SLOTS):
        store_wait(b)


@jax.jit
def kernel(bbox, x_table, y_table, h_table, w_table):
    bboxw = (bbox.astype(jnp.int32)
             .transpose(2, 0, 1)
             .reshape(NUM_K, _NW, N_CHUNKS, CHUNK)
             .transpose(1, 0, 2, 3)
             .reshape(_NW, NUM_K * N_CHUNKS, CHUNK))

    mesh = plsc.VectorSubcoreMesh(core_axis_name="c", subcore_axis_name="s")
    run = pl.kernel(
        _sc_body,
        out_type=jax.ShapeDtypeStruct((N_ROWS, HIDDEN), jnp.float32),
        mesh=mesh,
        scratch_types=[
            pltpu.VMEM((NUM_K * N_CHUNKS, CHUNK), jnp.int32),
            pltpu.VMEM_SHARED((NUM_K * MAX_POS, HIDDEN), jnp.float32),
        ] + [pltpu.VMEM((CHUNK, HIDDEN), jnp.float32)] * SLOTS
          + [pltpu.SemaphoreType.DMA] * (2 * SLOTS),
    )
    out = run(x_table, y_table, h_table, w_table, bboxw)
    return out.reshape(1024, 200, HIDDEN)
